# Initial kernel scaffold; baseline (speedup 1.0000x reference)
import jax
import jax.numpy as jnp
from jax.experimental import pallas as pl


def kernel(feat_static_cat, feat_static_real, past_time_feat, past_target, past_observed_values, future_time_feat, future_target, emb_table, W_embed, enc_Wqk, enc_Wv, enc_Wo, enc_W1, enc_W2, dec_Wqk, dec_Wv, dec_Wo, dec_W1, dec_W2, W_proj, b_proj):
    raise NotImplementedError("write your pallas kernel here")


# speedup vs baseline: 1.2358x; 1.2358x over previous
import jax
import jax.numpy as jnp
from jax.experimental import pallas as pl

CTX = 256
PRED = 64
LAGS = [1, 2, 3, 4, 5, 6, 7, 24, 48]
HIST = CTX + max(LAGS)
DM = 1024
NH = 16
DH = DM // NH
NE = 4
ND = 4
DFF = 4096
SUB = CTX + PRED

_F32 = jnp.float32


def _ln(x):
    m = jnp.mean(x, axis=-1, keepdims=True)
    v = jnp.mean((x - m) ** 2, axis=-1, keepdims=True)
    return (x - m) / jnp.sqrt(v + 1e-5)


def _mm(a, b):
    return jax.lax.dot_general(a, b, (((1,), (0,)), ((), ())),
                               preferred_element_type=_F32)


def _mm_t(a, b):
    return jax.lax.dot_general(a, b, (((1,), (1,)), ((), ())),
                               preferred_element_type=_F32)


def _softmax(s):
    s = s - jnp.max(s, axis=-1, keepdims=True)
    e = jnp.exp(s)
    return e / jnp.sum(e, axis=-1, keepdims=True)


def _embed_body(t_ref, w_ref, o_ref):
    o_ref[...] = _mm(t_ref[...], w_ref[...])


def _embed(ti_in, w_pad):
    m = ti_in.shape[0]
    blk = 2048
    return pl.pallas_call(
        _embed_body,
        grid=(m // blk,),
        in_specs=[pl.BlockSpec((blk, 128), lambda i: (i, 0)),
                  pl.BlockSpec((128, DM), lambda i: (0, 0))],
        out_specs=pl.BlockSpec((blk, DM), lambda i: (i, 0)),
        out_shape=jax.ShapeDtypeStruct((m, DM), _F32),
    )(ti_in, w_pad)


def _enc_attn_body(x_ref, wqk_ref, wv_ref, wo_ref, o_ref):
    x = x_ref[0]
    h = _ln(x)
    qk = _mm(h, wqk_ref[...])
    v = _mm(h, wv_ref[...])
    outs = []
    for hh in range(NH):
        sl = slice(hh * DH, (hh + 1) * DH)
        q = qk[:, sl]
        nrm = jnp.sqrt(jnp.sum(q * q, axis=-1, keepdims=True)) + 1e-8
        k = q / nrm
        s = _mm_t(q, k) * (1.0 / 8.0)
        a = _softmax(s)
        outs.append(_mm(a, v[:, sl]))
    o = jnp.concatenate(outs, axis=-1)
    o_ref[0] = x + _mm(o, wo_ref[...])


def _enc_attn(x, wqk, wv, wo):
    b = x.shape[0]
    return pl.pallas_call(
        _enc_attn_body,
        grid=(b,),
        in_specs=[pl.BlockSpec((1, CTX, DM), lambda i: (i, 0, 0)),
                  pl.BlockSpec((DM, DM), lambda i: (0, 0)),
                  pl.BlockSpec((DM, DM), lambda i: (0, 0)),
                  pl.BlockSpec((DM, DM), lambda i: (0, 0))],
        out_specs=pl.BlockSpec((1, CTX, DM), lambda i: (i, 0, 0)),
        out_shape=jax.ShapeDtypeStruct(x.shape, _F32),
    )(x, wqk, wv, wo)


def _ff_body(x_ref, w1_ref, w2_ref, o_ref):
    x = x_ref[0]
    h = _ln(x)
    t = jax.nn.gelu(_mm(h, w1_ref[...]))
    o_ref[0] = x + _mm(t, w2_ref[...])


def _ff(x, w1, w2, rows):
    b = x.shape[0]
    return pl.pallas_call(
        _ff_body,
        grid=(b,),
        in_specs=[pl.BlockSpec((1, rows, DM), lambda i: (i, 0, 0)),
                  pl.BlockSpec((DM, DFF), lambda i: (0, 0)),
                  pl.BlockSpec((DFF, DM), lambda i: (0, 0))],
        out_specs=pl.BlockSpec((1, rows, DM), lambda i: (i, 0, 0)),
        out_shape=jax.ShapeDtypeStruct(x.shape, _F32),
    )(x, w1, w2)


def _dec_attn_body(xe_ref, y_ref, wqk_ref, wv_ref, wo_ref, o_ref):
    xe = xe_ref[0]
    y = y_ref[0]
    ln_y = _ln(y)
    ln_kv = jnp.concatenate([_ln(xe), ln_y], axis=0)
    q = _mm(ln_y, wqk_ref[...])
    kr = _mm(ln_kv, wqk_ref[...])
    v = _mm(ln_kv, wv_ref[...])
    row = jax.lax.broadcasted_iota(jnp.int32, (PRED, CTX + PRED), 0)
    col = jax.lax.broadcasted_iota(jnp.int32, (PRED, CTX + PRED), 1)
    mask = (col < CTX) | ((col - CTX) <= row)
    outs = []
    for hh in range(NH):
        sl = slice(hh * DH, (hh + 1) * DH)
        kh = kr[:, sl]
        nrm = jnp.sqrt(jnp.sum(kh * kh, axis=-1, keepdims=True)) + 1e-8
        k = kh / nrm
        s = _mm_t(q[:, sl], k) * (1.0 / 8.0)
        s = jnp.where(mask, s, -1e9)
        a = _softmax(s)
        outs.append(_mm(a, v[:, sl]))
    o = jnp.concatenate(outs, axis=-1)
    o_ref[0] = y + _mm(o, wo_ref[...])


def _dec_attn(xe, y, wqk, wv, wo):
    b = y.shape[0]
    return pl.pallas_call(
        _dec_attn_body,
        grid=(b,),
        in_specs=[pl.BlockSpec((1, CTX, DM), lambda i: (i, 0, 0)),
                  pl.BlockSpec((1, PRED, DM), lambda i: (i, 0, 0)),
                  pl.BlockSpec((DM, DM), lambda i: (0, 0)),
                  pl.BlockSpec((DM, DM), lambda i: (0, 0)),
                  pl.BlockSpec((DM, DM), lambda i: (0, 0))],
        out_specs=pl.BlockSpec((1, PRED, DM), lambda i: (i, 0, 0)),
        out_shape=jax.ShapeDtypeStruct(y.shape, _F32),
    )(xe, y, wqk, wv, wo)


def _proj_body(y_ref, w_ref, b_ref, o_ref):
    raw = _mm(y_ref[...], w_ref[...]) + b_ref[...]
    col = jax.lax.broadcasted_iota(jnp.int32, raw.shape, 1)
    sp = jnp.maximum(raw, 0.0) + jnp.log1p(jnp.exp(-jnp.abs(raw)))
    o_ref[...] = jnp.where(col == 1, raw, sp) + jnp.where(col == 0, 2.0, 0.0)


def _proj(y2d, w_pad, b_pad):
    m = y2d.shape[0]
    return pl.pallas_call(
        _proj_body,
        grid=(1,),
        in_specs=[pl.BlockSpec((m, DM), lambda i: (0, 0)),
                  pl.BlockSpec((DM, 128), lambda i: (0, 0)),
                  pl.BlockSpec((1, 128), lambda i: (0, 0))],
        out_specs=pl.BlockSpec((m, 128), lambda i: (0, 0)),
        out_shape=jax.ShapeDtypeStruct((m, 128), _F32),
    )(y2d, w_pad, b_pad)


def kernel(feat_static_cat, feat_static_real, past_time_feat, past_target,
           past_observed_values, future_time_feat, future_target,
           emb_table, W_embed, enc_Wqk, enc_Wv, enc_Wo, enc_W1, enc_W2,
           dec_Wqk, dec_Wv, dec_Wo, dec_W1, dec_W2, W_proj, b_proj):
    bsz = past_target.shape[0]
    ctx = past_target[:, -CTX:]
    obs = past_observed_values[:, -CTX:]
    denom = jnp.clip(jnp.sum(obs, axis=1, keepdims=True), 1.0, None)
    loc = jnp.sum(ctx * obs, axis=1, keepdims=True) / denom
    var = jnp.sum(((ctx - loc) * obs) ** 2, axis=1, keepdims=True) / denom
    scale = jnp.sqrt(var + 1e-5)
    inputs = (jnp.concatenate([past_target, future_target], axis=1) - loc) / scale
    time_feat = jnp.concatenate(
        [past_time_feat[:, HIST - CTX:], future_time_feat], axis=1)
    emb = emb_table[feat_static_cat[:, 0]]
    log_abs_loc = jnp.sign(loc) * jnp.log1p(jnp.abs(loc))
    log_scale = jnp.log(scale)
    static = jnp.concatenate([emb, feat_static_real, log_abs_loc, log_scale],
                             axis=1)
    feats = jnp.concatenate([
        jnp.broadcast_to(static[:, None, :], (bsz, SUB, static.shape[-1])),
        time_feat], axis=-1)
    t_len = inputs.shape[1]
    lagged = jnp.stack(
        [inputs[:, t_len - l - SUB: t_len - l] for l in LAGS], axis=-1)
    ti_in = jnp.concatenate([lagged, feats], axis=-1)
    nin = ti_in.shape[-1]
    ti_pad = jnp.pad(ti_in, ((0, 0), (0, 0), (0, 128 - nin)))
    w_pad = jnp.pad(W_embed, ((0, 128 - nin), (0, 0)))
    ti = _embed(ti_pad.reshape(bsz * SUB, 128), w_pad).reshape(bsz, SUB, DM)

    x = ti[:, :CTX]
    y = ti[:, CTX:]
    for l in range(NE):
        x = _enc_attn(x, enc_Wqk[l], enc_Wv[l], enc_Wo[l])
        x = _ff(x, enc_W1[l], enc_W2[l], CTX)
    for l in range(ND):
        y = _dec_attn(x, y, dec_Wqk[l], dec_Wv[l], dec_Wo[l])
        y = _ff(y, dec_W1[l], dec_W2[l], PRED)

    wp_pad = jnp.pad(W_proj, ((0, 0), (0, 128 - W_proj.shape[-1])))
    bp_pad = jnp.pad(b_proj, ((0, 128 - b_proj.shape[0]),)).reshape(1, 128)
    out = _proj(y.reshape(bsz * PRED, DM), wp_pad, bp_pad)
    return out.reshape(bsz, PRED, 128)[:, :, :3]
